# Initial kernel scaffold; baseline (speedup 1.0000x reference)
import jax
import jax.numpy as jnp
from jax.experimental import pallas as pl


def kernel(tokens, old_to_new, U_hot, U_cold, B_hot, B_cold):
    raise NotImplementedError("write your pallas kernel here")


# speedup vs baseline: 7.2567x; 7.2567x over previous
import functools

import jax
import jax.numpy as jnp
from jax import lax
from jax.experimental import pallas as pl
from jax.experimental.pallas import tpu as pltpu
from jax.experimental.pallas import tpu_sc as plsc

_KHOT = 128
_RCOLD = 16
_D = 64
_NC, _NS, _L = 2, 16, 16
_NW = _NC * _NS


def _sc_gather(tokens_flat, old_to_new, U_cold):
    n = tokens_flat.shape[0]
    per_w = n // _NW
    mesh = plsc.VectorSubcoreMesh(core_axis_name="c", subcore_axis_name="s")

    @functools.partial(
        pl.kernel,
        out_type=(
            jax.ShapeDtypeStruct((n,), jnp.int32),
            jax.ShapeDtypeStruct((n, _RCOLD), jnp.float32),
        ),
        mesh=mesh,
        scratch_types=[
            pltpu.VMEM((per_w,), jnp.int32),
            pltpu.VMEM((per_w,), jnp.int32),
            pltpu.VMEM((per_w,), jnp.int32),
            pltpu.VMEM((per_w, _RCOLD), jnp.float32),
            pltpu.SemaphoreType.DMA,
        ],
        compiler_params=pltpu.CompilerParams(use_tc_tiling_on_sc=False),
    )
    def k(tok_hbm, o2n_hbm, ucold_hbm, newtok_hbm, rows_hbm,
          tok_v, nt_v, ci_v, rows_v, sem):
        wid = lax.axis_index("s") * _NC + lax.axis_index("c")
        base = wid * per_w
        pltpu.sync_copy(tok_hbm.at[pl.ds(base, per_w)], tok_v)
        pltpu.async_copy(o2n_hbm.at[tok_v], nt_v, sem).wait()

        def body(i, carry):
            nt = nt_v[pl.ds(i * _L, _L)]
            ci_v[pl.ds(i * _L, _L)] = jnp.maximum(nt - _KHOT, 0)
            return carry

        lax.fori_loop(0, per_w // _L, body, 0)
        pltpu.async_copy(ucold_hbm.at[ci_v], rows_v, sem).wait()
        pltpu.sync_copy(nt_v, newtok_hbm.at[pl.ds(base, per_w)])
        pltpu.sync_copy(rows_v, rows_hbm.at[pl.ds(base, per_w)])

    return k(tokens_flat, old_to_new, U_cold)


def _tc_body(nt_ref, rows_ref, uhot_ref, bhot_ref, bcold_ref, out_ref):
    nt = nt_ref[0, 0, :]
    cold = jnp.dot(rows_ref[...], bcold_ref[...],
                   preferred_element_type=jnp.float32)
    hot_tab = jnp.dot(uhot_ref[...], bhot_ref[...],
                      preferred_element_type=jnp.float32)
    ids = lax.broadcasted_iota(jnp.int32, (1, _KHOT), 1)
    onehot = (nt[:, None] == ids).astype(jnp.float32)
    hot = jnp.dot(onehot, hot_tab,
                  preferred_element_type=jnp.float32)
    is_hot = nt[:, None] < _KHOT
    out_ref[...] = jnp.where(is_hot, hot, cold)


def _tc_combine(new_tok, rows, U_hot, B_hot, B_cold):
    n = new_tok.shape[0]
    blk = 2048
    grid = n // blk
    nt3 = new_tok.reshape(grid, 1, blk)
    return pl.pallas_call(
        _tc_body,
        grid=(grid,),
        in_specs=[
            pl.BlockSpec((1, 1, blk), lambda i: (i, 0, 0)),
            pl.BlockSpec((blk, _RCOLD), lambda i: (i, 0)),
            pl.BlockSpec((_KHOT, _D), lambda i: (0, 0)),
            pl.BlockSpec((_D, _D), lambda i: (0, 0)),
            pl.BlockSpec((_RCOLD, _D), lambda i: (0, 0)),
        ],
        out_specs=pl.BlockSpec((blk, _D), lambda i: (i, 0)),
        out_shape=jax.ShapeDtypeStruct((n, _D), jnp.float32),
    )(nt3, rows, U_hot, B_hot, B_cold)


def kernel(tokens, old_to_new, U_hot, U_cold, B_hot, B_cold):
    tok_flat = tokens.reshape(-1)
    new_tok, rows = _sc_gather(tok_flat, old_to_new, U_cold)
    out = _tc_combine(new_tok, rows, U_hot, B_hot, B_cold)
    return out.reshape(tokens.shape + (_D,))
